# Initial kernel scaffold; baseline (speedup 1.0000x reference)
import jax
import jax.numpy as jnp
from jax.experimental import pallas as pl


def kernel(q, k, v, means, mem_key, mem_value):
    raise NotImplementedError("write your pallas kernel here")


# speedup vs baseline: 7.9195x; 7.9195x over previous
import functools

import jax
import jax.numpy as jnp
from jax import lax
from jax.experimental import pallas as pl
from jax.experimental.pallas import tpu as pltpu
from jax.experimental.pallas import tpu_sc as plsc

_B, _H, _T, _D = 2, 16, 4096, 64
_NC, _W = 32, 128
_BH = _B * _H
_R = _BH * _NC
_NWORK = 32
_RPW = _R // _NWORK


def _cumsum_lanes(x):
    n = 1
    t = x.shape[1]
    while n < t:
        x = x + jnp.pad(x, ((0, 0), (n, 0)))[:, :t]
        n *= 2
    return x


def _route_body(q_ref, k_ref, means_ref, mq_ref, mk_ref, den_ref, aux_ref):
    means = means_ref[0]
    msq = jnp.sum(means * means, axis=1)
    aux = jnp.float32(0.0)
    for half in range(2):
        src = q_ref if half == 0 else k_ref
        x = src[0]
        ssq = jnp.sum(x * x, axis=1, keepdims=True)
        xn = x / jnp.maximum(jnp.sqrt(ssq), 1e-12)
        dt = lax.dot_general(means, xn, (((1,), (1,)), ((), ())),
                             preferred_element_type=jnp.float32)
        mx = jnp.max(dt, axis=0)
        cio = lax.broadcasted_iota(jnp.int32, (_NC, _T), 0)
        amin = jnp.min(jnp.where(dt == mx[None, :], cio, _NC), axis=0)
        msel = jnp.sum(jnp.where(cio == amin[None, :], msq[:, None], 0.0), axis=0)
        xnsq = jnp.sum(xn * xn, axis=1)
        aux = aux + jnp.sum(xnsq) - 2.0 * jnp.sum(mx) + jnp.sum(msel)
        u = lax.bitcast_convert_type(dt, jnp.uint32)
        key = jnp.where(u >= jnp.uint32(0x80000000), ~u, u | jnp.uint32(0x80000000))
        kk = jnp.zeros((_NC, 1), jnp.uint32)
        for bit in range(31, -1, -1):
            cand = kk | jnp.uint32(1 << bit)
            cnt = jnp.sum((key >= cand).astype(jnp.float32), axis=1, keepdims=True)
            kk = jnp.where(cnt >= float(_W), cand, kk)
        gt = key > kk
        eq = key == kk
        n_gt = jnp.sum(gt.astype(jnp.int32), axis=1, keepdims=True)
        extra = _W - n_gt
        eqrank = _cumsum_lanes(eq.astype(jnp.int32))
        mask = gt | (eq & (eqrank <= extra))
        if half == 0:
            mq_ref[0] = mask.astype(jnp.int32)
            ones = jnp.ones((_NC, 1), jnp.float32)
            den_ref[0] = lax.dot_general(mask.astype(jnp.float32), ones,
                                         (((0,), (0,)), ((), ())),
                                         preferred_element_type=jnp.float32)
        else:
            mk_ref[0] = mask.astype(jnp.int32)
    aux_ref[0] = jnp.full((1, 128), aux, jnp.float32)


def _route_call(qf, kf, means):
    return pl.pallas_call(
        _route_body,
        grid=(_BH,),
        in_specs=[
            pl.BlockSpec((1, _T, _D), lambda i: (i, 0, 0)),
            pl.BlockSpec((1, _T, _D), lambda i: (i, 0, 0)),
            pl.BlockSpec((1, _NC, _D), lambda i: (i % _H, 0, 0)),
        ],
        out_specs=[
            pl.BlockSpec((1, _NC, _T), lambda i: (i, 0, 0)),
            pl.BlockSpec((1, _NC, _T), lambda i: (i, 0, 0)),
            pl.BlockSpec((1, _T, 1), lambda i: (i, 0, 0)),
            pl.BlockSpec((1, 1, 128), lambda i: (i, 0, 0)),
        ],
        out_shape=[
            jax.ShapeDtypeStruct((_BH, _NC, _T), jnp.int32),
            jax.ShapeDtypeStruct((_BH, _NC, _T), jnp.int32),
            jax.ShapeDtypeStruct((_BH, _T, 1), jnp.float32),
            jax.ShapeDtypeStruct((_BH, 1, 128), jnp.float32),
        ],
    )(qf, kf, means)


def _gather_call(mq, mk, qflat, kflat, vflat):
    mesh = plsc.VectorSubcoreMesh(core_axis_name="c", subcore_axis_name="s")

    @functools.partial(
        pl.kernel,
        mesh=mesh,
        out_type=[
            jax.ShapeDtypeStruct((_R, _W), jnp.int32),
            jax.ShapeDtypeStruct((_R, _W, 128), jnp.float32),
            jax.ShapeDtypeStruct((_R, _W, 128), jnp.float32),
            jax.ShapeDtypeStruct((_R, _W, 128), jnp.float32),
        ],
        scratch_types=[
            pltpu.VMEM((_T,), jnp.int32),
            pltpu.VMEM((_W + 16,), jnp.int32),
            pltpu.VMEM((_W,), jnp.int32),
            pltpu.VMEM((_W,), jnp.int32),
            pltpu.VMEM((_W, 128), jnp.float32),
            pltpu.SemaphoreType.DMA,
        ],
        compiler_params=pltpu.CompilerParams(needs_layout_passes=False),
    )
    def body(mq_hbm, mk_hbm, qf_hbm, kf_hbm, vf_hbm,
             idxq_hbm, qg_hbm, kg_hbm, vg_hbm,
             mask_v, buf_v, loc_v, glob_v, rows_v, sem):
        wid = lax.axis_index("s") * 2 + lax.axis_index("c")

        def compact(mask_hbm, r, base):
            pltpu.sync_copy(mask_hbm.at[r], mask_v)

            def chunk(i, off):
                mv = mask_v[pl.ds(i * 16, 16)]
                m = mv > 0
                idxs = i * 16 + lax.iota(jnp.int32, 16)
                pref = plsc.cumsum(mv)
                pos = off + pref - 1
                plsc.store_scatter(buf_v, [pos], idxs, mask=m)
                return off + jnp.sum(mv)

            lax.fori_loop(0, _T // 16, chunk, 0)
            for j in range(_W // 16):
                vv = buf_v[pl.ds(j * 16, 16)]
                loc_v[pl.ds(j * 16, 16)] = vv
                glob_v[pl.ds(j * 16, 16)] = vv + base

        def do_row(j, carry):
            r = wid * _RPW + j
            base = (r // _NC) * _T
            compact(mq_hbm, r, base)
            pltpu.sync_copy(loc_v, idxq_hbm.at[r])
            pltpu.async_copy(qf_hbm.at[glob_v], rows_v, sem).wait()
            pltpu.sync_copy(rows_v, qg_hbm.at[r])
            compact(mk_hbm, r, base)
            pltpu.async_copy(kf_hbm.at[glob_v], rows_v, sem).wait()
            pltpu.sync_copy(rows_v, kg_hbm.at[r])
            pltpu.async_copy(vf_hbm.at[glob_v], rows_v, sem).wait()
            pltpu.sync_copy(rows_v, vg_hbm.at[r])
            return carry

        lax.fori_loop(0, _RPW, do_row, 0)

    return body(mq, mk, qflat, kflat, vflat)


def _attn_body(qg_ref, kg_ref, vg_ref, mk_ref, mv_ref, bo_ref):
    qx = qg_ref[0][:, :_D]
    kx = kg_ref[0][:, :_D]
    vx = vg_ref[0][:, :_D]
    mk = mk_ref[0]
    mv = mv_ref[0]
    scale = jnp.float32(_D ** -0.5)
    dots = lax.dot_general(qx, kx, (((1,), (1,)), ((), ())),
                           preferred_element_type=jnp.float32) * scale
    dm = lax.dot_general(qx, mk, (((1,), (1,)), ((), ())),
                         preferred_element_type=jnp.float32) * scale
    m = jnp.maximum(jnp.max(dots, axis=1, keepdims=True), dm)
    ekv = jnp.exp(dots - m)
    em = jnp.exp(dm - m)
    den = jnp.sum(ekv, axis=1, keepdims=True) + em
    bo = (lax.dot_general(ekv, vx, (((1,), (0,)), ((), ())),
                          preferred_element_type=jnp.float32) + em * mv) / den
    bo_ref[0] = jnp.concatenate([bo, jnp.zeros((_W, 128 - _D), jnp.float32)], axis=1)


def _attn_call(qg, kg, vg, mem_key, mem_value):
    mk2 = mem_key.reshape(_H * _NC, 1, _D)
    mv2 = mem_value.reshape(_H * _NC, 1, _D)
    return pl.pallas_call(
        _attn_body,
        grid=(_R,),
        in_specs=[
            pl.BlockSpec((1, _W, 128), lambda i: (i, 0, 0)),
            pl.BlockSpec((1, _W, 128), lambda i: (i, 0, 0)),
            pl.BlockSpec((1, _W, 128), lambda i: (i, 0, 0)),
            pl.BlockSpec((1, 1, _D), lambda i: (i % (_H * _NC), 0, 0)),
            pl.BlockSpec((1, 1, _D), lambda i: (i % (_H * _NC), 0, 0)),
        ],
        out_specs=pl.BlockSpec((1, _W, 128), lambda i: (i, 0, 0)),
        out_shape=jax.ShapeDtypeStruct((_R, _W, 128), jnp.float32),
    )(qg, kg, vg, mk2, mv2)


def _scatter_call(bo4, idx3, zrows):
    mesh = plsc.VectorSubcoreMesh(core_axis_name="c", subcore_axis_name="s")

    @functools.partial(
        pl.kernel,
        mesh=mesh,
        out_type=jax.ShapeDtypeStruct((_BH * 16, 256, 128), jnp.float32),
        scratch_types=[
            pltpu.VMEM((2, _W, 128), jnp.float32),
            pltpu.VMEM((256, 128), jnp.float32),
            pltpu.VMEM((2, _W), jnp.int32),
            pltpu.VMEM_SHARED((_T, 128), jnp.float32),
        ],
        compiler_params=pltpu.CompilerParams(needs_layout_passes=False),
    )
    def body(bo_hbm, idx_hbm, z_hbm, numer_hbm, rows_sc, rows_lin, idx_v, accum):
        c = lax.axis_index("c")
        s = lax.axis_index("s")

        def do_bh(t, carry):
            bh = c * (_BH // 2) + t
            pltpu.sync_copy(z_hbm, rows_lin)
            pltpu.sync_copy(rows_lin, accum.at[pl.ds(s * 256, 256)])
            plsc.subcore_barrier()
            pltpu.sync_copy(bo_hbm.at[bh, pl.ds(s * 2, 2)], rows_sc)
            pltpu.sync_copy(idx_hbm.at[bh, pl.ds(s * 2, 2)], idx_v)
            for u in range(2):
                pltpu.sync_copy(rows_sc.at[u], accum.at[idx_v.at[u]], add=True)
            plsc.subcore_barrier()
            pltpu.sync_copy(accum.at[pl.ds(s * 256, 256)], rows_lin)
            pltpu.sync_copy(rows_lin, numer_hbm.at[bh * 16 + s])
            return carry

        lax.fori_loop(0, _BH // 2, do_bh, 0)

    return body(bo4, idx3, zrows)


def _div_body(num_ref, den_ref, o_ref):
    o_ref[0] = num_ref[0][:, :_D] / (den_ref[0] + 1e-5)


def _div_call(numer, den):
    return pl.pallas_call(
        _div_body,
        grid=(_BH,),
        in_specs=[
            pl.BlockSpec((1, _T, 128), lambda i: (i, 0, 0)),
            pl.BlockSpec((1, _T, 1), lambda i: (i, 0, 0)),
        ],
        out_specs=pl.BlockSpec((1, _T, _D), lambda i: (i, 0, 0)),
        out_shape=jax.ShapeDtypeStruct((_BH, _T, _D), jnp.float32),
    )(numer, den)


def kernel(q, k, v, means, mem_key, mem_value):
    qf = q.reshape(_BH, _T, _D)
    kf = k.reshape(_BH, _T, _D)
    vf = v.reshape(_BH, _T, _D)
    mq, mk_, den, auxp = _route_call(qf, kf, means)
    aux_loss = jnp.sum(auxp[:, 0, 0]) / float(_BH * 2 * _T * _D)
    pad = ((0, 0), (0, 128 - _D))
    idxq, qg, kg, vg = _gather_call(
        mq.reshape(_R, _T), mk_.reshape(_R, _T),
        jnp.pad(qf.reshape(_BH * _T, _D), pad),
        jnp.pad(kf.reshape(_BH * _T, _D), pad),
        jnp.pad(vf.reshape(_BH * _T, _D), pad))
    bo = _attn_call(qg, kg, vg, mem_key, mem_value)
    zrows = jnp.zeros((256, 128), jnp.float32)
    numer = _scatter_call(bo.reshape(_BH, _NC, _W, 128),
                          idxq.reshape(_BH, _NC, _W), zrows)
    out = _div_call(numer.reshape(_BH, _T, 128), den)
    return out.reshape(_B, _H, _T, _D), aux_loss


# speedup vs baseline: 8.3382x; 1.0529x over previous
import functools

import jax
import jax.numpy as jnp
from jax import lax
from jax.experimental import pallas as pl
from jax.experimental.pallas import tpu as pltpu
from jax.experimental.pallas import tpu_sc as plsc

_B, _H, _T, _D = 2, 16, 4096, 64
_NC, _W = 32, 128
_BH = _B * _H
_R = _BH * _NC
_NWORK = 32
_RPW = _R // _NWORK


def _cumsum_lanes(x):
    n = 1
    t = x.shape[1]
    while n < t:
        x = x + jnp.pad(x, ((0, 0), (n, 0)))[:, :t]
        n *= 2
    return x


def _route_body(q_ref, k_ref, means_ref, mq_ref, mk_ref, den_ref, aux_ref):
    means = means_ref[0]
    msq = jnp.sum(means * means, axis=1)
    aux = jnp.float32(0.0)
    for half in range(2):
        src = q_ref if half == 0 else k_ref
        x = src[0]
        ssq = jnp.sum(x * x, axis=1, keepdims=True)
        xn = x / jnp.maximum(jnp.sqrt(ssq), 1e-12)
        dt = lax.dot_general(means, xn, (((1,), (1,)), ((), ())),
                             preferred_element_type=jnp.float32)
        mx = jnp.max(dt, axis=0)
        cio = lax.broadcasted_iota(jnp.int32, (_NC, _T), 0)
        amin = jnp.min(jnp.where(dt == mx[None, :], cio, _NC), axis=0)
        msel = jnp.sum(jnp.where(cio == amin[None, :], msq[:, None], 0.0), axis=0)
        xnsq = jnp.sum(xn * xn, axis=1)
        aux = aux + jnp.sum(xnsq) - 2.0 * jnp.sum(mx) + jnp.sum(msel)
        u = lax.bitcast_convert_type(dt, jnp.uint32)
        key = jnp.where(u >= jnp.uint32(0x80000000), ~u, u | jnp.uint32(0x80000000))
        kk = jnp.zeros((_NC, 1), jnp.uint32)
        for bit in range(31, -1, -1):
            cand = kk | jnp.uint32(1 << bit)
            cnt = jnp.sum((key >= cand).astype(jnp.float32), axis=1, keepdims=True)
            kk = jnp.where(cnt >= float(_W), cand, kk)
        gt = key > kk
        eq = key == kk
        n_gt = jnp.sum(gt.astype(jnp.int32), axis=1, keepdims=True)
        extra = _W - n_gt
        eqrank = _cumsum_lanes(eq.astype(jnp.int32))
        mask = gt | (eq & (eqrank <= extra))
        pos = _cumsum_lanes(mask.astype(jnp.int32)) - 1
        posm = jnp.where(mask, pos, -1)
        if half == 0:
            mq_ref[0] = posm
            ones = jnp.ones((_NC, 1), jnp.float32)
            den_ref[0] = lax.dot_general(mask.astype(jnp.float32), ones,
                                         (((0,), (0,)), ((), ())),
                                         preferred_element_type=jnp.float32)
        else:
            mk_ref[0] = posm
    aux_ref[0] = jnp.full((1, 128), aux, jnp.float32)


def _route_call(qf, kf, means):
    return pl.pallas_call(
        _route_body,
        grid=(_BH,),
        in_specs=[
            pl.BlockSpec((1, _T, _D), lambda i: (i, 0, 0)),
            pl.BlockSpec((1, _T, _D), lambda i: (i, 0, 0)),
            pl.BlockSpec((1, _NC, _D), lambda i: (i % _H, 0, 0)),
        ],
        out_specs=[
            pl.BlockSpec((1, _NC, _T), lambda i: (i, 0, 0)),
            pl.BlockSpec((1, _NC, _T), lambda i: (i, 0, 0)),
            pl.BlockSpec((1, _T, 1), lambda i: (i, 0, 0)),
            pl.BlockSpec((1, 1, 128), lambda i: (i, 0, 0)),
        ],
        out_shape=[
            jax.ShapeDtypeStruct((_BH, _NC, _T), jnp.int32),
            jax.ShapeDtypeStruct((_BH, _NC, _T), jnp.int32),
            jax.ShapeDtypeStruct((_BH, _T, 1), jnp.float32),
            jax.ShapeDtypeStruct((_BH, 1, 128), jnp.float32),
        ],
    )(qf, kf, means)


def _gather_call(mq, mk, qflat, kflat, vflat):
    mesh = plsc.VectorSubcoreMesh(core_axis_name="c", subcore_axis_name="s")

    @functools.partial(
        pl.kernel,
        mesh=mesh,
        out_type=[
            jax.ShapeDtypeStruct((_R, _W), jnp.int32),
            jax.ShapeDtypeStruct((_R, _W, 128), jnp.float32),
            jax.ShapeDtypeStruct((_R, _W, 128), jnp.float32),
            jax.ShapeDtypeStruct((_R, _W, 128), jnp.float32),
        ],
        scratch_types=[
            pltpu.VMEM((_T,), jnp.int32),
            pltpu.VMEM((_W,), jnp.int32),
            pltpu.VMEM((_W,), jnp.int32),
            pltpu.VMEM((_W,), jnp.int32),
            pltpu.VMEM((_W, 128), jnp.float32),
            pltpu.VMEM((_W, 128), jnp.float32),
            pltpu.VMEM((_W, 128), jnp.float32),
            pltpu.SemaphoreType.DMA,
            pltpu.SemaphoreType.DMA,
            pltpu.SemaphoreType.DMA,
        ],
        compiler_params=pltpu.CompilerParams(needs_layout_passes=False),
    )
    def body(mq_hbm, mk_hbm, qf_hbm, kf_hbm, vf_hbm,
             idxq_hbm, qg_hbm, kg_hbm, vg_hbm,
             pm_v, loc_v, gq_v, gk_v, rq_v, rk_v, rv_v, sem1, sem2, sem3):
        wid = lax.axis_index("s") * 2 + lax.axis_index("c")

        def compact(pm_hbm, r, dst_v, extra_v):
            pltpu.sync_copy(pm_hbm.at[r], pm_v)

            def chunk(i, carry):
                pm = pm_v[pl.ds(i * 16, 16)]
                m = pm >= 0
                idxs = i * 16 + lax.iota(jnp.int32, 16)
                plsc.store_scatter(dst_v, [pm], idxs, mask=m)
                return carry

            lax.fori_loop(0, _T // 16, chunk, 0)

        def do_row(j, carry):
            r = wid * _RPW + j
            base = (r // _NC) * _T
            compact(mq_hbm, r, loc_v, gq_v)
            for u in range(_W // 16):
                gq_v[pl.ds(u * 16, 16)] = loc_v[pl.ds(u * 16, 16)] + base
            cq = pltpu.async_copy(qf_hbm.at[gq_v], rq_v, sem1)
            pltpu.sync_copy(loc_v, idxq_hbm.at[r])
            compact(mk_hbm, r, gk_v, None)
            for u in range(_W // 16):
                gk_v[pl.ds(u * 16, 16)] = gk_v[pl.ds(u * 16, 16)] + base
            ck = pltpu.async_copy(kf_hbm.at[gk_v], rk_v, sem2)
            cv = pltpu.async_copy(vf_hbm.at[gk_v], rv_v, sem3)
            cq.wait()
            pltpu.sync_copy(rq_v, qg_hbm.at[r])
            ck.wait()
            pltpu.sync_copy(rk_v, kg_hbm.at[r])
            cv.wait()
            pltpu.sync_copy(rv_v, vg_hbm.at[r])
            return carry

        lax.fori_loop(0, _RPW, do_row, 0)

    return body(mq, mk, qflat, kflat, vflat)


def _attn_body(qg_ref, kg_ref, vg_ref, mk_ref, mv_ref, bo_ref):
    qx = qg_ref[0][:, :_D]
    kx = kg_ref[0][:, :_D]
    vx = vg_ref[0][:, :_D]
    mk = mk_ref[0]
    mv = mv_ref[0]
    scale = jnp.float32(_D ** -0.5)
    dots = lax.dot_general(qx, kx, (((1,), (1,)), ((), ())),
                           preferred_element_type=jnp.float32) * scale
    dm = lax.dot_general(qx, mk, (((1,), (1,)), ((), ())),
                         preferred_element_type=jnp.float32) * scale
    m = jnp.maximum(jnp.max(dots, axis=1, keepdims=True), dm)
    ekv = jnp.exp(dots - m)
    em = jnp.exp(dm - m)
    den = jnp.sum(ekv, axis=1, keepdims=True) + em
    bo = (lax.dot_general(ekv, vx, (((1,), (0,)), ((), ())),
                          preferred_element_type=jnp.float32) + em * mv) / den
    bo_ref[0] = jnp.concatenate([bo, jnp.zeros((_W, 128 - _D), jnp.float32)], axis=1)


def _attn_call(qg, kg, vg, mem_key, mem_value):
    mk2 = mem_key.reshape(_H * _NC, 1, _D)
    mv2 = mem_value.reshape(_H * _NC, 1, _D)
    return pl.pallas_call(
        _attn_body,
        grid=(_R,),
        in_specs=[
            pl.BlockSpec((1, _W, 128), lambda i: (i, 0, 0)),
            pl.BlockSpec((1, _W, 128), lambda i: (i, 0, 0)),
            pl.BlockSpec((1, _W, 128), lambda i: (i, 0, 0)),
            pl.BlockSpec((1, 1, _D), lambda i: (i % (_H * _NC), 0, 0)),
            pl.BlockSpec((1, 1, _D), lambda i: (i % (_H * _NC), 0, 0)),
        ],
        out_specs=pl.BlockSpec((1, _W, 128), lambda i: (i, 0, 0)),
        out_shape=jax.ShapeDtypeStruct((_R, _W, 128), jnp.float32),
    )(qg, kg, vg, mk2, mv2)


def _scatter_call(bo4, idx3, zrows):
    mesh = plsc.VectorSubcoreMesh(core_axis_name="c", subcore_axis_name="s")

    @functools.partial(
        pl.kernel,
        mesh=mesh,
        out_type=jax.ShapeDtypeStruct((_BH * 16, 256, 128), jnp.float32),
        scratch_types=[
            pltpu.VMEM((2, _W, 128), jnp.float32),
            pltpu.VMEM((256, 128), jnp.float32),
            pltpu.VMEM((2, _W), jnp.int32),
            pltpu.VMEM_SHARED((_T, 128), jnp.float32),
        ],
        compiler_params=pltpu.CompilerParams(needs_layout_passes=False),
    )
    def body(bo_hbm, idx_hbm, z_hbm, numer_hbm, rows_sc, rows_lin, idx_v, accum):
        c = lax.axis_index("c")
        s = lax.axis_index("s")

        def do_bh(t, carry):
            bh = c * (_BH // 2) + t
            pltpu.sync_copy(z_hbm, rows_lin)
            pltpu.sync_copy(rows_lin, accum.at[pl.ds(s * 256, 256)])
            plsc.subcore_barrier()
            pltpu.sync_copy(bo_hbm.at[bh, pl.ds(s * 2, 2)], rows_sc)
            pltpu.sync_copy(idx_hbm.at[bh, pl.ds(s * 2, 2)], idx_v)
            for u in range(2):
                pltpu.sync_copy(rows_sc.at[u], accum.at[idx_v.at[u]], add=True)
            plsc.subcore_barrier()
            pltpu.sync_copy(accum.at[pl.ds(s * 256, 256)], rows_lin)
            pltpu.sync_copy(rows_lin, numer_hbm.at[bh * 16 + s])
            return carry

        lax.fori_loop(0, _BH // 2, do_bh, 0)

    return body(bo4, idx3, zrows)


def _div_body(num_ref, den_ref, o_ref):
    o_ref[0] = num_ref[0][:, :_D] / (den_ref[0] + 1e-5)


def _div_call(numer, den):
    return pl.pallas_call(
        _div_body,
        grid=(_BH,),
        in_specs=[
            pl.BlockSpec((1, _T, 128), lambda i: (i, 0, 0)),
            pl.BlockSpec((1, _T, 1), lambda i: (i, 0, 0)),
        ],
        out_specs=pl.BlockSpec((1, _T, _D), lambda i: (i, 0, 0)),
        out_shape=jax.ShapeDtypeStruct((_BH, _T, _D), jnp.float32),
    )(numer, den)


def kernel(q, k, v, means, mem_key, mem_value):
    qf = q.reshape(_BH, _T, _D)
    kf = k.reshape(_BH, _T, _D)
    vf = v.reshape(_BH, _T, _D)
    mq, mk_, den, auxp = _route_call(qf, kf, means)
    aux_loss = jnp.sum(auxp[:, 0, 0]) / float(_BH * 2 * _T * _D)
    pad = ((0, 0), (0, 128 - _D))
    idxq, qg, kg, vg = _gather_call(
        mq.reshape(_R, _T), mk_.reshape(_R, _T),
        jnp.pad(qf.reshape(_BH * _T, _D), pad),
        jnp.pad(kf.reshape(_BH * _T, _D), pad),
        jnp.pad(vf.reshape(_BH * _T, _D), pad))
    bo = _attn_call(qg, kg, vg, mem_key, mem_value)
    zrows = jnp.zeros((256, 128), jnp.float32)
    numer = _scatter_call(bo.reshape(_BH, _NC, _W, 128),
                          idxq.reshape(_BH, _NC, _W), zrows)
    out = _div_call(numer.reshape(_BH, _T, 128), den)
    return out.reshape(_B, _H, _T, _D), aux_loss


# speedup vs baseline: 11.4295x; 1.3707x over previous
import functools

import jax
import jax.numpy as jnp
from jax import lax
from jax.experimental import pallas as pl
from jax.experimental.pallas import tpu as pltpu
from jax.experimental.pallas import tpu_sc as plsc

_B, _H, _T, _D = 2, 16, 4096, 64
_NC, _W = 32, 128
_BH = _B * _H
_R = _BH * _NC
_NWORK = 32
_RPW = _R // _NWORK


def _cumsum_lanes(x):
    n = 1
    t = x.shape[1]
    while n < t:
        x = x + jnp.pad(x, ((0, 0), (n, 0)))[:, :t]
        n *= 2
    return x


def _route_body(q_ref, k_ref, means_ref, mq_ref, mk_ref, den_ref, aux_ref):
    means = means_ref[0]
    msq = jnp.sum(means * means, axis=1)
    aux = jnp.float32(0.0)
    for half in range(2):
        src = q_ref if half == 0 else k_ref
        x = src[0]
        ssq = jnp.sum(x * x, axis=1, keepdims=True)
        xn = x / jnp.maximum(jnp.sqrt(ssq), 1e-12)
        dt = lax.dot_general(means, xn, (((1,), (1,)), ((), ())),
                             preferred_element_type=jnp.float32)
        mx = jnp.max(dt, axis=0)
        cio = lax.broadcasted_iota(jnp.int32, (_NC, _T), 0)
        amin = jnp.min(jnp.where(dt == mx[None, :], cio, _NC), axis=0)
        msel = jnp.sum(jnp.where(cio == amin[None, :], msq[:, None], 0.0), axis=0)
        xnsq = jnp.sum(xn * xn, axis=1)
        aux = aux + jnp.sum(xnsq) - 2.0 * jnp.sum(mx) + jnp.sum(msel)
        u = lax.bitcast_convert_type(dt, jnp.uint32)
        key = jnp.where(u >= jnp.uint32(0x80000000), ~u, u | jnp.uint32(0x80000000))
        kk = jnp.zeros((_NC, 1), jnp.uint32)
        for bit in range(31, -1, -1):
            cand = kk | jnp.uint32(1 << bit)
            cnt = jnp.sum((key >= cand).astype(jnp.float32), axis=1, keepdims=True)
            kk = jnp.where(cnt >= float(_W), cand, kk)
        gt = key > kk
        eq = key == kk
        n_gt = jnp.sum(gt.astype(jnp.int32), axis=1, keepdims=True)
        extra = _W - n_gt
        eqrank = _cumsum_lanes(eq.astype(jnp.int32))
        mask = gt | (eq & (eqrank <= extra))
        pos = _cumsum_lanes(mask.astype(jnp.int32)) - 1
        posm = jnp.where(mask, pos, -1)
        if half == 0:
            mq_ref[0] = posm
            ones = jnp.ones((_NC, 1), jnp.float32)
            den_ref[0] = lax.dot_general(mask.astype(jnp.float32), ones,
                                         (((0,), (0,)), ((), ())),
                                         preferred_element_type=jnp.float32)
        else:
            mk_ref[0] = posm
    aux_ref[0] = jnp.full((1, 128), aux, jnp.float32)


def _route_call(qf, kf, means):
    return pl.pallas_call(
        _route_body,
        grid=(_BH,),
        in_specs=[
            pl.BlockSpec((1, _T, _D), lambda i: (i, 0, 0)),
            pl.BlockSpec((1, _T, _D), lambda i: (i, 0, 0)),
            pl.BlockSpec((1, _NC, _D), lambda i: (i % _H, 0, 0)),
        ],
        out_specs=[
            pl.BlockSpec((1, _NC, _T), lambda i: (i, 0, 0)),
            pl.BlockSpec((1, _NC, _T), lambda i: (i, 0, 0)),
            pl.BlockSpec((1, _T, 1), lambda i: (i, 0, 0)),
            pl.BlockSpec((1, 1, 128), lambda i: (i, 0, 0)),
        ],
        out_shape=[
            jax.ShapeDtypeStruct((_BH, _NC, _T), jnp.int32),
            jax.ShapeDtypeStruct((_BH, _NC, _T), jnp.int32),
            jax.ShapeDtypeStruct((_BH, _T, 1), jnp.float32),
            jax.ShapeDtypeStruct((_BH, 1, 128), jnp.float32),
        ],
    )(qf, kf, means)


def _gather_call(mq, mk, qflat, kflat, vflat):
    mesh = plsc.VectorSubcoreMesh(core_axis_name="c", subcore_axis_name="s")

    @functools.partial(
        pl.kernel,
        mesh=mesh,
        out_type=[
            jax.ShapeDtypeStruct((_R, _W), jnp.int32),
            jax.ShapeDtypeStruct((_R, _W, 128), jnp.float32),
            jax.ShapeDtypeStruct((_R, _W, 128), jnp.float32),
            jax.ShapeDtypeStruct((_R, _W, 128), jnp.float32),
        ],
        scratch_types=[
            pltpu.VMEM((_T,), jnp.int32),
            pltpu.VMEM((_W,), jnp.int32),
            pltpu.VMEM((_W,), jnp.int32),
            pltpu.VMEM((_W,), jnp.int32),
            pltpu.VMEM((_W, 128), jnp.float32),
            pltpu.VMEM((_W, 128), jnp.float32),
            pltpu.VMEM((_W, 128), jnp.float32),
            pltpu.SemaphoreType.DMA,
            pltpu.SemaphoreType.DMA,
            pltpu.SemaphoreType.DMA,
        ],
        compiler_params=pltpu.CompilerParams(needs_layout_passes=False),
    )
    def body(mq_hbm, mk_hbm, qf_hbm, kf_hbm, vf_hbm,
             idxq_hbm, qg_hbm, kg_hbm, vg_hbm,
             pm_v, loc_v, gq_v, gk_v, rq_v, rk_v, rv_v, sem1, sem2, sem3):
        wid = lax.axis_index("s") * 2 + lax.axis_index("c")

        def compact(pm_hbm, r, dst_v, extra_v):
            pltpu.sync_copy(pm_hbm.at[r], pm_v)

            def chunk(i, carry):
                pm = pm_v[pl.ds(i * 16, 16)]
                m = pm >= 0
                idxs = i * 16 + lax.iota(jnp.int32, 16)
                plsc.store_scatter(dst_v, [pm], idxs, mask=m)
                return carry

            lax.fori_loop(0, _T // 16, chunk, 0)

        def do_row(j, carry):
            r = wid * _RPW + j
            base = (r // _NC) * _T
            compact(mq_hbm, r, loc_v, gq_v)
            for u in range(_W // 16):
                gq_v[pl.ds(u * 16, 16)] = loc_v[pl.ds(u * 16, 16)] + base
            cq = pltpu.async_copy(qf_hbm.at[gq_v], rq_v, sem1)
            pltpu.sync_copy(loc_v, idxq_hbm.at[r])
            compact(mk_hbm, r, gk_v, None)
            for u in range(_W // 16):
                gk_v[pl.ds(u * 16, 16)] = gk_v[pl.ds(u * 16, 16)] + base
            ck = pltpu.async_copy(kf_hbm.at[gk_v], rk_v, sem2)
            cv = pltpu.async_copy(vf_hbm.at[gk_v], rv_v, sem3)
            cq.wait()
            pltpu.sync_copy(rq_v, qg_hbm.at[r])
            ck.wait()
            pltpu.sync_copy(rk_v, kg_hbm.at[r])
            cv.wait()
            pltpu.sync_copy(rv_v, vg_hbm.at[r])
            return carry

        lax.fori_loop(0, _RPW, do_row, 0)

    return body(mq, mk, qflat, kflat, vflat)


_G = 8


def _attn_body(qg_ref, kg_ref, vg_ref, mk_ref, mv_ref, bo_ref):
    scale = jnp.float32(_D ** -0.5)
    for g in range(_G):
        qx = qg_ref[g]
        kx = kg_ref[g]
        vx = vg_ref[g]
        mk = mk_ref[g]
        mv = mv_ref[g]
        dots = lax.dot_general(qx, kx, (((1,), (1,)), ((), ())),
                               preferred_element_type=jnp.float32) * scale
        dm = lax.dot_general(qx, mk, (((1,), (1,)), ((), ())),
                             preferred_element_type=jnp.float32) * scale
        m = jnp.maximum(jnp.max(dots, axis=1, keepdims=True), dm)
        ekv = jnp.exp(dots - m)
        em = jnp.exp(dm - m)
        den = jnp.sum(ekv, axis=1, keepdims=True) + em
        bo_ref[g] = (lax.dot_general(ekv, vx, (((1,), (0,)), ((), ())),
                                     preferred_element_type=jnp.float32) + em * mv) / den


def _attn_call(qg, kg, vg, mem_key, mem_value):
    pad3 = ((0, 0), (0, 0), (0, 128 - _D))
    mk2 = jnp.pad(mem_key.reshape(_H * _NC, 1, _D), pad3)
    mv2 = jnp.pad(mem_value.reshape(_H * _NC, 1, _D), pad3)
    nmem_blocks = (_H * _NC) // _G
    return pl.pallas_call(
        _attn_body,
        grid=(_R // _G,),
        in_specs=[
            pl.BlockSpec((_G, _W, 128), lambda i: (i, 0, 0)),
            pl.BlockSpec((_G, _W, 128), lambda i: (i, 0, 0)),
            pl.BlockSpec((_G, _W, 128), lambda i: (i, 0, 0)),
            pl.BlockSpec((_G, 1, 128), lambda i: (i % nmem_blocks, 0, 0)),
            pl.BlockSpec((_G, 1, 128), lambda i: (i % nmem_blocks, 0, 0)),
        ],
        out_specs=pl.BlockSpec((_G, _W, 128), lambda i: (i, 0, 0)),
        out_shape=jax.ShapeDtypeStruct((_R, _W, 128), jnp.float32),
    )(qg, kg, vg, mk2, mv2)


def _scatter_call(bo4, idx3, zrows):
    mesh = plsc.VectorSubcoreMesh(core_axis_name="c", subcore_axis_name="s")

    @functools.partial(
        pl.kernel,
        mesh=mesh,
        out_type=jax.ShapeDtypeStruct((_BH * 32, _W, 128), jnp.float32),
        scratch_types=[
            pltpu.VMEM((2, _W, 128), jnp.float32),
            pltpu.VMEM((2, _W), jnp.int32),
            pltpu.VMEM((256, 128), jnp.float32),
            pltpu.VMEM_SHARED((_T, 128), jnp.float32),
        ],
        compiler_params=pltpu.CompilerParams(needs_layout_passes=False),
    )
    def body(bo_hbm, idx_hbm, z_hbm, numer_hbm, rows_sc, idx_v, zero_v, accum):
        c = lax.axis_index("c")
        s = lax.axis_index("s")
        pltpu.sync_copy(z_hbm, zero_v)

        def do_bh(t, carry):
            bh = c * (_BH // 2) + t
            pltpu.sync_copy(zero_v, accum.at[pl.ds(s * 256, 256)])
            plsc.subcore_barrier()
            pltpu.sync_copy(bo_hbm.at[bh, pl.ds(s * 2, 2)], rows_sc)
            pltpu.sync_copy(idx_hbm.at[bh, pl.ds(s * 2, 2)], idx_v)
            for u in range(2):
                pltpu.sync_copy(rows_sc.at[u], accum.at[idx_v.at[u]], add=True)
            plsc.subcore_barrier()
            for u in range(2):
                pltpu.sync_copy(accum.at[pl.ds(s * 256 + u * _W, _W)], rows_sc.at[u])
                pltpu.sync_copy(rows_sc.at[u], numer_hbm.at[(bh * 16 + s) * 2 + u])
            return carry

        lax.fori_loop(0, _BH // 2, do_bh, 0)

    return body(bo4, idx3, zrows)


def _div_body(num_ref, den_ref, o_ref):
    o_ref[0] = num_ref[0][:, :_D] / (den_ref[0] + 1e-5)


def _div_call(numer, den):
    return pl.pallas_call(
        _div_body,
        grid=(_BH,),
        in_specs=[
            pl.BlockSpec((1, _T, 128), lambda i: (i, 0, 0)),
            pl.BlockSpec((1, _T, 1), lambda i: (i, 0, 0)),
        ],
        out_specs=pl.BlockSpec((1, _T, _D), lambda i: (i, 0, 0)),
        out_shape=jax.ShapeDtypeStruct((_BH, _T, _D), jnp.float32),
    )(numer, den)


def kernel(q, k, v, means, mem_key, mem_value):
    qf = q.reshape(_BH, _T, _D)
    kf = k.reshape(_BH, _T, _D)
    vf = v.reshape(_BH, _T, _D)
    mq, mk_, den, auxp = _route_call(qf, kf, means)
    aux_loss = jnp.sum(auxp[:, 0, 0]) / float(_BH * 2 * _T * _D)
    pad = ((0, 0), (0, 128 - _D))
    idxq, qg, kg, vg = _gather_call(
        mq.reshape(_R, _T), mk_.reshape(_R, _T),
        jnp.pad(qf.reshape(_BH * _T, _D), pad),
        jnp.pad(kf.reshape(_BH * _T, _D), pad),
        jnp.pad(vf.reshape(_BH * _T, _D), pad))
    bo = _attn_call(qg, kg, vg, mem_key, mem_value)
    zrows = jnp.zeros((256, 128), jnp.float32)
    numer = _scatter_call(bo.reshape(_BH, _NC, _W, 128),
                          idxq.reshape(_BH, _NC, _W), zrows)
    out = _div_call(numer.reshape(_BH, _T, 128), den)
    return out.reshape(_B, _H, _T, _D), aux_loss


# speedup vs baseline: 11.6633x; 1.0205x over previous
import functools

import jax
import jax.numpy as jnp
from jax import lax
from jax.experimental import pallas as pl
from jax.experimental.pallas import tpu as pltpu
from jax.experimental.pallas import tpu_sc as plsc

_B, _H, _T, _D = 2, 16, 4096, 64
_NC, _W = 32, 128
_BH = _B * _H
_R = _BH * _NC
_NWORK = 32
_RPW = _R // _NWORK


def _cumsum_lanes(x):
    n = 1
    t = x.shape[1]
    while n < t:
        x = x + jnp.pad(x, ((0, 0), (n, 0)))[:, :t]
        n *= 2
    return x


def _route_body(q_ref, k_ref, means_ref, mq_ref, mk_ref, den_ref, aux_ref):
    means = means_ref[0]
    msq = jnp.sum(means * means, axis=1)
    aux = jnp.float32(0.0)
    for half in range(2):
        src = q_ref if half == 0 else k_ref
        x = src[0]
        ssq = jnp.sum(x * x, axis=1, keepdims=True)
        xn = x / jnp.maximum(jnp.sqrt(ssq), 1e-12)
        dt = lax.dot_general(means, xn, (((1,), (1,)), ((), ())),
                             preferred_element_type=jnp.float32)
        mx = jnp.max(dt, axis=0)
        cio = lax.broadcasted_iota(jnp.int32, (_NC, _T), 0)
        amin = jnp.min(jnp.where(dt == mx[None, :], cio, _NC), axis=0)
        msel = jnp.sum(jnp.where(cio == amin[None, :], msq[:, None], 0.0), axis=0)
        xnsq = jnp.sum(xn * xn, axis=1)
        aux = aux + jnp.sum(xnsq) - 2.0 * jnp.sum(mx) + jnp.sum(msel)
        u = lax.bitcast_convert_type(dt, jnp.uint32)
        key = jnp.where(u >= jnp.uint32(0x80000000), ~u, u | jnp.uint32(0x80000000))
        kk = jnp.zeros((_NC, 1), jnp.uint32)
        for bit in range(31, -1, -1):
            cand = kk | jnp.uint32(1 << bit)
            cnt = jnp.sum((key >= cand).astype(jnp.float32), axis=1, keepdims=True)
            kk = jnp.where(cnt >= float(_W), cand, kk)
        gt = key > kk
        eq = key == kk
        n_gt = jnp.sum(gt.astype(jnp.int32), axis=1, keepdims=True)
        extra = _W - n_gt
        eqrank = _cumsum_lanes(eq.astype(jnp.int32))
        mask = gt | (eq & (eqrank <= extra))
        pos = _cumsum_lanes(mask.astype(jnp.int32)) - 1
        posm = jnp.where(mask, pos, -1)
        if half == 0:
            mq_ref[0] = posm
            ones = jnp.ones((_NC, 1), jnp.float32)
            den_ref[0] = lax.dot_general(mask.astype(jnp.float32), ones,
                                         (((0,), (0,)), ((), ())),
                                         preferred_element_type=jnp.float32)
        else:
            mk_ref[0] = posm
    aux_ref[0] = jnp.full((1, 128), aux, jnp.float32)


def _route_call(qf, kf, means):
    return pl.pallas_call(
        _route_body,
        grid=(_BH,),
        in_specs=[
            pl.BlockSpec((1, _T, _D), lambda i: (i, 0, 0)),
            pl.BlockSpec((1, _T, _D), lambda i: (i, 0, 0)),
            pl.BlockSpec((1, _NC, _D), lambda i: (i % _H, 0, 0)),
        ],
        out_specs=[
            pl.BlockSpec((1, _NC, _T), lambda i: (i, 0, 0)),
            pl.BlockSpec((1, _NC, _T), lambda i: (i, 0, 0)),
            pl.BlockSpec((1, _T, 1), lambda i: (i, 0, 0)),
            pl.BlockSpec((1, 1, 128), lambda i: (i, 0, 0)),
        ],
        out_shape=[
            jax.ShapeDtypeStruct((_BH, _NC, _T), jnp.int32),
            jax.ShapeDtypeStruct((_BH, _NC, _T), jnp.int32),
            jax.ShapeDtypeStruct((_BH, _T, 1), jnp.float32),
            jax.ShapeDtypeStruct((_BH, 1, 128), jnp.float32),
        ],
    )(qf, kf, means)


def _gather_call(mq, mk, qflat, kflat, vflat):
    mesh = plsc.VectorSubcoreMesh(core_axis_name="c", subcore_axis_name="s")

    @functools.partial(
        pl.kernel,
        mesh=mesh,
        out_type=[
            jax.ShapeDtypeStruct((_R, _W), jnp.int32),
            jax.ShapeDtypeStruct((_R, _W, 128), jnp.float32),
            jax.ShapeDtypeStruct((_R, _W, 128), jnp.float32),
            jax.ShapeDtypeStruct((_R, _W, 128), jnp.float32),
        ],
        scratch_types=[
            pltpu.VMEM((2 * _T,), jnp.int32),
            pltpu.VMEM((2 * _T,), jnp.int32),
            pltpu.VMEM((_W,), jnp.int32),
            pltpu.VMEM((_W,), jnp.int32),
            pltpu.VMEM((_W,), jnp.int32),
            pltpu.VMEM((_W, 128), jnp.float32),
            pltpu.VMEM((_W, 128), jnp.float32),
            pltpu.VMEM((_W, 128), jnp.float32),
            pltpu.SemaphoreType.DMA,
            pltpu.SemaphoreType.DMA,
            pltpu.SemaphoreType.DMA,
            pltpu.SemaphoreType.DMA,
            pltpu.SemaphoreType.DMA,
        ],
        compiler_params=pltpu.CompilerParams(needs_layout_passes=False),
    )
    def body(mq_hbm, mk_hbm, qf_hbm, kf_hbm, vf_hbm,
             idxq_hbm, qg_hbm, kg_hbm, vg_hbm,
             pmq_v, pmk_v, loc_v, gq_v, gk_v, rq_v, rk_v, rv_v,
             sem1, sem2, sem3, semq, semk):
        wid = lax.axis_index("s") * 2 + lax.axis_index("c")
        r0 = wid * _RPW
        rlast = r0 + _RPW - 1

        def compact(pm_buf, off, dst_v):
            def chunk(i, carry):
                pm = pm_buf[pl.ds(off + i * 16, 16)]
                m = pm >= 0
                idxs = i * 16 + lax.iota(jnp.int32, 16)
                plsc.store_scatter(dst_v, [pm], idxs, mask=m)
                return carry

            lax.fori_loop(0, _T // 16, chunk, 0)

        pltpu.async_copy(mq_hbm.at[r0], pmq_v.at[pl.ds(0, _T)], semq)
        pltpu.async_copy(mk_hbm.at[r0], pmk_v.at[pl.ds(0, _T)], semk)

        def do_row(j, carry):
            r = r0 + j
            sl = lax.rem(j, 2)
            nsl = lax.rem(j + 1, 2)
            rn = jnp.minimum(r + 1, rlast)
            base = (r // _NC) * _T
            pltpu.make_async_copy(mq_hbm.at[r], pmq_v.at[pl.ds(sl * _T, _T)], semq).wait()
            compact(pmq_v, sl * _T, loc_v)
            for u in range(_W // 16):
                gq_v[pl.ds(u * 16, 16)] = loc_v[pl.ds(u * 16, 16)] + base
            pltpu.async_copy(qf_hbm.at[gq_v], rq_v, sem1)
            pltpu.sync_copy(loc_v, idxq_hbm.at[r])
            pltpu.make_async_copy(mk_hbm.at[r], pmk_v.at[pl.ds(sl * _T, _T)], semk).wait()
            compact(pmk_v, sl * _T, gk_v)
            for u in range(_W // 16):
                gk_v[pl.ds(u * 16, 16)] = gk_v[pl.ds(u * 16, 16)] + base
            pltpu.async_copy(kf_hbm.at[gk_v], rk_v, sem2)
            pltpu.async_copy(vf_hbm.at[gk_v], rv_v, sem3)
            pltpu.async_copy(mq_hbm.at[rn], pmq_v.at[pl.ds(nsl * _T, _T)], semq)
            pltpu.async_copy(mk_hbm.at[rn], pmk_v.at[pl.ds(nsl * _T, _T)], semk)
            pltpu.make_async_copy(qf_hbm.at[gq_v], rq_v, sem1).wait()
            pltpu.sync_copy(rq_v, qg_hbm.at[r])
            pltpu.make_async_copy(kf_hbm.at[gk_v], rk_v, sem2).wait()
            pltpu.sync_copy(rk_v, kg_hbm.at[r])
            pltpu.make_async_copy(vf_hbm.at[gk_v], rv_v, sem3).wait()
            pltpu.sync_copy(rv_v, vg_hbm.at[r])
            return carry

        lax.fori_loop(0, _RPW, do_row, 0)
        pltpu.make_async_copy(mq_hbm.at[rlast], pmq_v.at[pl.ds((_RPW % 2) * _T, _T)], semq).wait()
        pltpu.make_async_copy(mk_hbm.at[rlast], pmk_v.at[pl.ds((_RPW % 2) * _T, _T)], semk).wait()

    return body(mq, mk, qflat, kflat, vflat)


_G = 8


def _attn_body(qg_ref, kg_ref, vg_ref, mk_ref, mv_ref, bo_ref):
    scale = jnp.float32(_D ** -0.5)
    for g in range(_G):
        qx = qg_ref[g]
        kx = kg_ref[g]
        vx = vg_ref[g]
        mk = mk_ref[g]
        mv = mv_ref[g]
        dots = lax.dot_general(qx, kx, (((1,), (1,)), ((), ())),
                               preferred_element_type=jnp.float32) * scale
        dm = lax.dot_general(qx, mk, (((1,), (1,)), ((), ())),
                             preferred_element_type=jnp.float32) * scale
        m = jnp.maximum(jnp.max(dots, axis=1, keepdims=True), dm)
        ekv = jnp.exp(dots - m)
        em = jnp.exp(dm - m)
        den = jnp.sum(ekv, axis=1, keepdims=True) + em
        bo_ref[g] = (lax.dot_general(ekv, vx, (((1,), (0,)), ((), ())),
                                     preferred_element_type=jnp.float32) + em * mv) / den


def _attn_call(qg, kg, vg, mem_key, mem_value):
    pad3 = ((0, 0), (0, 0), (0, 128 - _D))
    mk2 = jnp.pad(mem_key.reshape(_H * _NC, 1, _D), pad3)
    mv2 = jnp.pad(mem_value.reshape(_H * _NC, 1, _D), pad3)
    nmem_blocks = (_H * _NC) // _G
    return pl.pallas_call(
        _attn_body,
        grid=(_R // _G,),
        in_specs=[
            pl.BlockSpec((_G, _W, 128), lambda i: (i, 0, 0)),
            pl.BlockSpec((_G, _W, 128), lambda i: (i, 0, 0)),
            pl.BlockSpec((_G, _W, 128), lambda i: (i, 0, 0)),
            pl.BlockSpec((_G, 1, 128), lambda i: (i % nmem_blocks, 0, 0)),
            pl.BlockSpec((_G, 1, 128), lambda i: (i % nmem_blocks, 0, 0)),
        ],
        out_specs=pl.BlockSpec((_G, _W, 128), lambda i: (i, 0, 0)),
        out_shape=jax.ShapeDtypeStruct((_R, _W, 128), jnp.float32),
    )(qg, kg, vg, mk2, mv2)


def _scatter_call(bo4, idx3, zrows):
    mesh = plsc.VectorSubcoreMesh(core_axis_name="c", subcore_axis_name="s")

    @functools.partial(
        pl.kernel,
        mesh=mesh,
        out_type=jax.ShapeDtypeStruct((_BH * 32, _W, 128), jnp.float32),
        scratch_types=[
            pltpu.VMEM((2, _W, 128), jnp.float32),
            pltpu.VMEM((2, _W), jnp.int32),
            pltpu.VMEM((256, 128), jnp.float32),
            pltpu.VMEM_SHARED((_T, 128), jnp.float32),
        ],
        compiler_params=pltpu.CompilerParams(needs_layout_passes=False),
    )
    def body(bo_hbm, idx_hbm, z_hbm, numer_hbm, rows_sc, idx_v, zero_v, accum):
        c = lax.axis_index("c")
        s = lax.axis_index("s")
        pltpu.sync_copy(z_hbm, zero_v)

        def do_bh(t, carry):
            bh = c * (_BH // 2) + t
            pltpu.sync_copy(zero_v, accum.at[pl.ds(s * 256, 256)])
            plsc.subcore_barrier()
            pltpu.sync_copy(bo_hbm.at[bh, pl.ds(s * 2, 2)], rows_sc)
            pltpu.sync_copy(idx_hbm.at[bh, pl.ds(s * 2, 2)], idx_v)
            for u in range(2):
                pltpu.sync_copy(rows_sc.at[u], accum.at[idx_v.at[u]], add=True)
            plsc.subcore_barrier()
            for u in range(2):
                pltpu.sync_copy(accum.at[pl.ds(s * 256 + u * _W, _W)], rows_sc.at[u])
                pltpu.sync_copy(rows_sc.at[u], numer_hbm.at[(bh * 16 + s) * 2 + u])
            return carry

        lax.fori_loop(0, _BH // 2, do_bh, 0)

    return body(bo4, idx3, zrows)


def _div_body(num_ref, den_ref, o_ref):
    o_ref[0] = num_ref[0][:, :_D] / (den_ref[0] + 1e-5)


def _div_call(numer, den):
    return pl.pallas_call(
        _div_body,
        grid=(_BH,),
        in_specs=[
            pl.BlockSpec((1, _T, 128), lambda i: (i, 0, 0)),
            pl.BlockSpec((1, _T, 1), lambda i: (i, 0, 0)),
        ],
        out_specs=pl.BlockSpec((1, _T, _D), lambda i: (i, 0, 0)),
        out_shape=jax.ShapeDtypeStruct((_BH, _T, _D), jnp.float32),
    )(numer, den)


def kernel(q, k, v, means, mem_key, mem_value):
    qf = q.reshape(_BH, _T, _D)
    kf = k.reshape(_BH, _T, _D)
    vf = v.reshape(_BH, _T, _D)
    mq, mk_, den, auxp = _route_call(qf, kf, means)
    aux_loss = jnp.sum(auxp[:, 0, 0]) / float(_BH * 2 * _T * _D)
    pad = ((0, 0), (0, 128 - _D))
    idxq, qg, kg, vg = _gather_call(
        mq.reshape(_R, _T), mk_.reshape(_R, _T),
        jnp.pad(qf.reshape(_BH * _T, _D), pad),
        jnp.pad(kf.reshape(_BH * _T, _D), pad),
        jnp.pad(vf.reshape(_BH * _T, _D), pad))
    bo = _attn_call(qg, kg, vg, mem_key, mem_value)
    zrows = jnp.zeros((256, 128), jnp.float32)
    numer = _scatter_call(bo.reshape(_BH, _NC, _W, 128),
                          idxq.reshape(_BH, _NC, _W), zrows)
    out = _div_call(numer.reshape(_BH, _T, 128), den)
    return out.reshape(_B, _H, _T, _D), aux_loss
